# Initial kernel scaffold; baseline (speedup 1.0000x reference)
import jax
import jax.numpy as jnp
from jax.experimental import pallas as pl


def kernel(x, hm_w1, hm_b1, hm_g, hm_beta, hm_mean, hm_var, hm_w2, hm_b2, vec_w1, vec_b1, vec_g, vec_beta, vec_mean, vec_var, vec_w2, vec_b2):
    raise NotImplementedError("write your pallas kernel here")


# speedup vs baseline: 3.4289x; 3.4289x over previous
import functools

import jax
import jax.numpy as jnp
from jax import lax
from jax.experimental import pallas as pl
from jax.experimental.pallas import tpu as pltpu

B, C, H, W = 4, 32, 512, 512
HEAD = 64
RADIUS = 5
NUM_PEAKS = 18
THRESH = 0.1

R = 64
NR = H // R
OC = 2 * HEAD


def _conv_kernel(x_ref, halo_ref, a_ref, b1_ref, w2_ref, b2_ref,
                 hm_ref, vec_ref, strip, s_r, s_l, im2):
    strip[:, 1:R + 1, :] = x_ref[0]
    strip[:, 0, :] = halo_ref[0, 0, 0]
    strip[:, R + 1, :] = halo_ref[0, 0, 1]
    sv = strip[...]
    s_r[...] = jnp.pad(sv[:, :, :W - 1], ((0, 0), (0, 0), (1, 0)))
    s_l[...] = jnp.pad(sv[:, :, 1:], ((0, 0), (0, 0), (0, 1)))

    def group(g, _):
        yb = pl.multiple_of(g * 8, 8)
        v = strip[:, pl.ds(yb, 10), :]
        vr = s_r[:, pl.ds(yb, 10), :]
        vl = s_l[:, pl.ds(yb, 10), :]
        outs = []
        for r in range(8):
            for dy in range(3):
                im2[3 * dy + 0, :, :] = vr[:, r + dy, :]
                im2[3 * dy + 1, :, :] = v[:, r + dy, :]
                im2[3 * dy + 2, :, :] = vl[:, r + dy, :]
            col = im2[...].reshape(9 * C, W)
            h = jnp.dot(a_ref[...], col, preferred_element_type=jnp.float32)
            h = jnp.maximum(h + b1_ref[...], 0.0)
            o = jnp.dot(w2_ref[...], h, preferred_element_type=jnp.float32)
            outs.append(o + b2_ref[...])
        ostk = jnp.stack(outs, axis=1)
        hm_ref[0, :, pl.ds(yb, 8), :] = jax.nn.sigmoid(ostk[0:2])
        vec_ref[0, :, pl.ds(yb, 8), :] = ostk[2:4]
        return 0

    lax.fori_loop(0, R // 8, group, 0)


def _heads(x, a, b1, w2, b2):
    z = jnp.zeros((B, C, 1, W), jnp.float32)
    tops = jnp.concatenate([z, x[:, :, R - 1:H - 1:R, :]], axis=2)
    bots = jnp.concatenate([x[:, :, R:H:R, :], z], axis=2)
    halo = jnp.stack([tops, bots], axis=2)
    halo = jnp.transpose(halo, (0, 3, 2, 1, 4))

    grid = (B, NR)
    out = pl.pallas_call(
        _conv_kernel,
        grid=grid,
        in_specs=[
            pl.BlockSpec((1, C, R, W), lambda b, j: (b, 0, j, 0)),
            pl.BlockSpec((1, 1, 2, C, W), lambda b, j: (b, j, 0, 0, 0)),
            pl.BlockSpec((OC, 9 * C), lambda b, j: (0, 0)),
            pl.BlockSpec((OC, 1), lambda b, j: (0, 0)),
            pl.BlockSpec((8, OC), lambda b, j: (0, 0)),
            pl.BlockSpec((8, 1), lambda b, j: (0, 0)),
        ],
        out_specs=[
            pl.BlockSpec((1, 2, R, W), lambda b, j: (b, 0, j, 0)),
            pl.BlockSpec((1, 2, R, W), lambda b, j: (b, 0, j, 0)),
        ],
        out_shape=[
            jax.ShapeDtypeStruct((B, 2, H, W), jnp.float32),
            jax.ShapeDtypeStruct((B, 2, H, W), jnp.float32),
        ],
        scratch_shapes=[
            pltpu.VMEM((C, R + 2, W), jnp.float32),
            pltpu.VMEM((C, R + 2, W), jnp.float32),
            pltpu.VMEM((C, R + 2, W), jnp.float32),
            pltpu.VMEM((9, C, W), jnp.float32),
        ],
        compiler_params=pltpu.CompilerParams(
            dimension_semantics=("parallel", "parallel")),
    )(x, halo, a, b1, w2, b2)
    return out


WIN = 24


def _peaks_kernel(hm_ref, out_ref, hms, rmax):
    hm0 = hm_ref[0]
    hms[...] = hm0
    rmax[...] = jnp.max(hm0, axis=1, keepdims=True)

    iota_w = lax.broadcasted_iota(jnp.int32, (1, W), 1)
    iota_h = lax.broadcasted_iota(jnp.int32, (H, 1), 0)
    iota_8 = lax.broadcasted_iota(jnp.int32, (8, W), 0)
    big = jnp.int32(1 << 30)

    def body(i, peaks):
        rm = rmax[...]
        m = jnp.max(rm)
        y = jnp.min(jnp.where(rm == m, iota_h, big))
        ry = pl.multiple_of((y // 8) * 8, 8)
        rows8 = hms[pl.ds(ry, 8), :]
        rowv = jnp.max(jnp.where(iota_8 == (y - ry), rows8, -1.0),
                       axis=0, keepdims=True)
        x = jnp.min(jnp.where(rowv == m, iota_w, big))
        valid = m > THRESH

        gb = jnp.clip((y - RADIUS) // 8, 0, (H - WIN) // 8)
        yb = pl.multiple_of(gb * 8, 8)
        win = hms[pl.ds(yb, WIN), :]
        rr = yb + lax.broadcasted_iota(jnp.int32, (WIN, W), 0)
        cc = lax.broadcasted_iota(jnp.int32, (WIN, W), 1)
        sup = ((jnp.abs(rr - y) <= RADIUS) & (jnp.abs(cc - x) <= RADIUS)
               & valid)
        neww = jnp.where(sup, 0.0, win)
        hms[pl.ds(yb, WIN), :] = neww
        rmax[pl.ds(yb, WIN), :] = jnp.max(neww, axis=1, keepdims=True)

        rowm = lax.broadcasted_iota(jnp.int32, (NUM_PEAKS, 2), 0) == i
        colm = lax.broadcasted_iota(jnp.int32, (NUM_PEAKS, 2), 1)
        new = jnp.where(colm == 0, x.astype(jnp.float32),
                        y.astype(jnp.float32))
        return jnp.where(rowm & valid, new, peaks)

    peaks = lax.fori_loop(0, NUM_PEAKS, body,
                          jnp.zeros((NUM_PEAKS, 2), jnp.float32))

    px = peaks[:, 0]
    py = peaks[:, 1]
    validk = (px + py) != 0.0
    key = jnp.where(validk, py, -jnp.inf)
    jj = lax.broadcasted_iota(jnp.int32, (NUM_PEAKS, NUM_PEAKS), 0)
    ii = lax.broadcasted_iota(jnp.int32, (NUM_PEAKS, NUM_PEAKS), 1)
    kj = key[:, None]
    ki = key[None, :]
    rank = jnp.sum(((kj > ki) | ((kj == ki) & (jj < ii))).astype(jnp.int32),
                   axis=0)
    ptsm = jnp.where(validk[:, None], peaks, 0.0)
    onehot = (rank[:, None] ==
              lax.broadcasted_iota(jnp.int32, (NUM_PEAKS, NUM_PEAKS), 1))
    ordered = jnp.sum(onehot.astype(jnp.float32)[:, :, None]
                      * ptsm[:, None, :], axis=0)
    out_ref[0] = jnp.pad(ordered, ((0, 24 - NUM_PEAKS), (0, 126)))


def _extract_all_peaks(dual_hm):
    hm8 = dual_hm.reshape(2 * B, H, W)
    out = pl.pallas_call(
        _peaks_kernel,
        grid=(2 * B,),
        in_specs=[pl.BlockSpec((1, H, W), lambda i: (i, 0, 0))],
        out_specs=pl.BlockSpec((1, 24, 128), lambda i: (i, 0, 0)),
        out_shape=jax.ShapeDtypeStruct((2 * B, 24, 128), jnp.float32),
        scratch_shapes=[
            pltpu.VMEM((H, W), jnp.float32),
            pltpu.VMEM((H, 1), jnp.float32),
        ],
        compiler_params=pltpu.CompilerParams(
            dimension_semantics=("arbitrary",)),
    )(hm8)
    pk = out.reshape(B, 2, 24, 128)
    return pk[:, 0, :NUM_PEAKS, :2], pk[:, 1, :NUM_PEAKS, :2]


def kernel(x, hm_w1, hm_b1, hm_g, hm_beta, hm_mean, hm_var, hm_w2, hm_b2,
           vec_w1, vec_b1, vec_g, vec_beta, vec_mean, vec_var, vec_w2,
           vec_b2):
    eps = 1e-5
    s_hm = hm_g / jnp.sqrt(hm_var + eps)
    s_vec = vec_g / jnp.sqrt(vec_var + eps)
    w1 = jnp.concatenate([hm_w1 * s_hm[:, None, None, None],
                          vec_w1 * s_vec[:, None, None, None]], axis=0)
    a = jnp.transpose(w1, (0, 2, 3, 1)).reshape(OC, 9 * C)
    b1 = jnp.concatenate([hm_b1 * s_hm + (hm_beta - hm_mean * s_hm),
                          vec_b1 * s_vec + (vec_beta - vec_mean * s_vec)])
    b1 = b1[:, None]
    w2 = jnp.zeros((8, OC), jnp.float32)
    w2 = w2.at[0:2, 0:HEAD].set(hm_w2.reshape(2, HEAD))
    w2 = w2.at[2:4, HEAD:OC].set(vec_w2.reshape(2, HEAD))
    b2 = jnp.zeros((8, 1), jnp.float32)
    b2 = b2.at[0:2, 0].set(hm_b2)
    b2 = b2.at[2:4, 0].set(vec_b2)

    dual_hm, vec_ind = _heads(x, a, b1, w2, b2)
    ordered_upper, ordered_lower = _extract_all_peaks(dual_hm)
    mid = (ordered_upper + ordered_lower) / 2.0
    return (dual_hm, ordered_upper, ordered_lower, mid, vec_ind)


# speedup vs baseline: 3.4682x; 1.0115x over previous
import functools

import jax
import jax.numpy as jnp
from jax import lax
from jax.experimental import pallas as pl
from jax.experimental.pallas import tpu as pltpu
from jax.experimental.pallas import tpu_sc as plsc

B, C, H, W = 4, 32, 512, 512
HEAD = 64
RADIUS = 5
NUM_PEAKS = 18
THRESH = 0.1

R = 64
NR = H // R
OC = 2 * HEAD


def _conv_kernel(x_ref, halo_ref, a_ref, b1_ref, w2_ref, b2_ref,
                 hm_ref, vec_ref, strip, s_r, s_l, im2):
    strip[:, 1:R + 1, :] = x_ref[0]
    strip[:, 0, :] = halo_ref[0, 0, 0]
    strip[:, R + 1, :] = halo_ref[0, 0, 1]
    sv = strip[...]
    s_r[...] = jnp.pad(sv[:, :, :W - 1], ((0, 0), (0, 0), (1, 0)))
    s_l[...] = jnp.pad(sv[:, :, 1:], ((0, 0), (0, 0), (0, 1)))

    def group(g, _):
        yb = pl.multiple_of(g * 8, 8)
        v = strip[:, pl.ds(yb, 10), :]
        vr = s_r[:, pl.ds(yb, 10), :]
        vl = s_l[:, pl.ds(yb, 10), :]
        outs = []
        for r in range(8):
            for dy in range(3):
                im2[3 * dy + 0, :, :] = vr[:, r + dy, :]
                im2[3 * dy + 1, :, :] = v[:, r + dy, :]
                im2[3 * dy + 2, :, :] = vl[:, r + dy, :]
            col = im2[...].reshape(9 * C, W)
            h = jnp.dot(a_ref[...], col, preferred_element_type=jnp.float32)
            h = jnp.maximum(h + b1_ref[...], 0.0)
            o = jnp.dot(w2_ref[...], h, preferred_element_type=jnp.float32)
            outs.append(o + b2_ref[...])
        ostk = jnp.stack(outs, axis=1)
        hm_ref[0, :, pl.ds(yb, 8), :] = jax.nn.sigmoid(ostk[0:2])
        vec_ref[0, :, pl.ds(yb, 8), :] = ostk[2:4]
        return 0

    lax.fori_loop(0, R // 8, group, 0)


def _heads(x, a, b1, w2, b2):
    z = jnp.zeros((B, C, 1, W), jnp.float32)
    tops = jnp.concatenate([z, x[:, :, R - 1:H - 1:R, :]], axis=2)
    bots = jnp.concatenate([x[:, :, R:H:R, :], z], axis=2)
    halo = jnp.stack([tops, bots], axis=2)
    halo = jnp.transpose(halo, (0, 3, 2, 1, 4))

    grid = (B, NR)
    out = pl.pallas_call(
        _conv_kernel,
        grid=grid,
        in_specs=[
            pl.BlockSpec((1, C, R, W), lambda b, j: (b, 0, j, 0)),
            pl.BlockSpec((1, 1, 2, C, W), lambda b, j: (b, j, 0, 0, 0)),
            pl.BlockSpec((OC, 9 * C), lambda b, j: (0, 0)),
            pl.BlockSpec((OC, 1), lambda b, j: (0, 0)),
            pl.BlockSpec((8, OC), lambda b, j: (0, 0)),
            pl.BlockSpec((8, 1), lambda b, j: (0, 0)),
        ],
        out_specs=[
            pl.BlockSpec((1, 2, R, W), lambda b, j: (b, 0, j, 0)),
            pl.BlockSpec((1, 2, R, W), lambda b, j: (b, 0, j, 0)),
        ],
        out_shape=[
            jax.ShapeDtypeStruct((B, 2, H, W), jnp.float32),
            jax.ShapeDtypeStruct((B, 2, H, W), jnp.float32),
        ],
        scratch_shapes=[
            pltpu.VMEM((C, R + 2, W), jnp.float32),
            pltpu.VMEM((C, R + 2, W), jnp.float32),
            pltpu.VMEM((C, R + 2, W), jnp.float32),
            pltpu.VMEM((9, C, W), jnp.float32),
        ],
        compiler_params=pltpu.CompilerParams(
            dimension_semantics=("parallel", "parallel")),
    )(x, halo, a, b1, w2, b2)
    return out


WIN = 24


def _peaks_kernel(hm_ref, out_ref, hms, rmax):
    hm0 = hm_ref[0]
    hms[...] = hm0
    rmax[...] = jnp.max(hm0, axis=1, keepdims=True)

    iota_w = lax.broadcasted_iota(jnp.int32, (1, W), 1)
    iota_h = lax.broadcasted_iota(jnp.int32, (H, 1), 0)
    iota_8 = lax.broadcasted_iota(jnp.int32, (8, W), 0)
    big = jnp.int32(1 << 30)

    def body(i, peaks):
        rm = rmax[...]
        m = jnp.max(rm)
        y = jnp.min(jnp.where(rm == m, iota_h, big))
        ry = pl.multiple_of((y // 8) * 8, 8)
        rows8 = hms[pl.ds(ry, 8), :]
        rowv = jnp.max(jnp.where(iota_8 == (y - ry), rows8, -1.0),
                       axis=0, keepdims=True)
        x = jnp.min(jnp.where(rowv == m, iota_w, big))
        valid = m > THRESH

        gb = jnp.clip((y - RADIUS) // 8, 0, (H - WIN) // 8)
        yb = pl.multiple_of(gb * 8, 8)
        win = hms[pl.ds(yb, WIN), :]
        rr = yb + lax.broadcasted_iota(jnp.int32, (WIN, W), 0)
        cc = lax.broadcasted_iota(jnp.int32, (WIN, W), 1)
        sup = ((jnp.abs(rr - y) <= RADIUS) & (jnp.abs(cc - x) <= RADIUS)
               & valid)
        neww = jnp.where(sup, 0.0, win)
        hms[pl.ds(yb, WIN), :] = neww
        rmax[pl.ds(yb, WIN), :] = jnp.max(neww, axis=1, keepdims=True)

        rowm = lax.broadcasted_iota(jnp.int32, (NUM_PEAKS, 2), 0) == i
        colm = lax.broadcasted_iota(jnp.int32, (NUM_PEAKS, 2), 1)
        new = jnp.where(colm == 0, x.astype(jnp.float32),
                        y.astype(jnp.float32))
        return jnp.where(rowm & valid, new, peaks)

    peaks = lax.fori_loop(0, NUM_PEAKS, body,
                          jnp.zeros((NUM_PEAKS, 2), jnp.float32))

    px = peaks[:, 0]
    py = peaks[:, 1]
    validk = (px + py) != 0.0
    key = jnp.where(validk, py, -jnp.inf)
    jj = lax.broadcasted_iota(jnp.int32, (NUM_PEAKS, NUM_PEAKS), 0)
    ii = lax.broadcasted_iota(jnp.int32, (NUM_PEAKS, NUM_PEAKS), 1)
    kj = key[:, None]
    ki = key[None, :]
    rank = jnp.sum(((kj > ki) | ((kj == ki) & (jj < ii))).astype(jnp.int32),
                   axis=0)
    ptsm = jnp.where(validk[:, None], peaks, 0.0)
    onehot = (rank[:, None] ==
              lax.broadcasted_iota(jnp.int32, (NUM_PEAKS, NUM_PEAKS), 1))
    ordered = jnp.sum(onehot.astype(jnp.float32)[:, :, None]
                      * ptsm[:, None, :], axis=0)
    out_ref[0] = jnp.pad(ordered, ((0, 24 - NUM_PEAKS), (0, 126)))


def _extract_all_peaks(dual_hm):
    hm8 = dual_hm.reshape(2 * B, H, W)
    out = pl.pallas_call(
        _peaks_kernel,
        grid=(2 * B,),
        in_specs=[pl.BlockSpec((1, H, W), lambda i: (i, 0, 0))],
        out_specs=pl.BlockSpec((1, 24, 128), lambda i: (i, 0, 0)),
        out_shape=jax.ShapeDtypeStruct((2 * B, 24, 128), jnp.float32),
        scratch_shapes=[
            pltpu.VMEM((H, W), jnp.float32),
            pltpu.VMEM((H, 1), jnp.float32),
        ],
        compiler_params=pltpu.CompilerParams(
            dimension_semantics=("arbitrary",)),
    )(hm8)
    pk = out.reshape(B, 2, 24, 128)
    return pk[:, 0, :NUM_PEAKS, :2], pk[:, 1, :NUM_PEAKS, :2]


SROWS = H // 4
NEG = -3.0e38


def _vmax_s(v):
    m = v[0]
    for l in range(1, 16):
        m = jnp.maximum(m, v[l])
    return m


def _vmin_s(v):
    m = v[0]
    for l in range(1, 16):
        m = jnp.minimum(m, v[l])
    return m


def _sc_peaks_kernel(hm_hbm, out_hbm, strip, rmaxv, pub, grp, pkv, rdv,
                     ox, oy, shared_max, shared_pk, sm):
    c = lax.axis_index("c")
    s = lax.axis_index("s")
    g = s // 4
    local = s % 4
    r0 = local * SROWS
    m = c * 4 + g

    i16 = lax.broadcasted_iota(jnp.int32, (16,), 0)
    big = jnp.int32(1 << 30)
    negv = jnp.full((16,), NEG, jnp.float32)

    pltpu.sync_copy(hm_hbm.at[m, pl.ds(r0, SROWS), :], strip)

    def _row_max(r):
        def chunk(k, acc):
            v = strip[pl.ds(r, 1), pl.ds(k * 16, 16)].reshape((16,))
            return jnp.maximum(acc, v)
        return _vmax_s(lax.fori_loop(0, W // 16, chunk, negv))

    def init_group(rg, _):
        def row_in(j, accv):
            ms = _row_max(rg * 16 + j)
            return jnp.where(i16 == j, ms, accv)
        rmaxv[pl.ds(rg * 16, 16)] = lax.fori_loop(
            0, 16, row_in, jnp.zeros((16,), jnp.float32))
        return 0

    lax.fori_loop(0, SROWS // 16, init_group, 0)

    def iter_body(i, _):
        def chunk8(k, acc):
            return jnp.maximum(acc, rmaxv[pl.ds(k * 16, 16)])
        sm_t = _vmax_s(lax.fori_loop(0, SROWS // 16, chunk8, negv))
        pub[...] = jnp.full((16,), sm_t, jnp.float32)
        pltpu.sync_copy(pub, shared_max.at[pl.ds(s * 16, 16)])
        plsc.subcore_barrier()

        pltpu.sync_copy(shared_max, grp)
        m0 = grp[pl.ds((g * 4 + 0) * 16, 16)][0]
        m1 = grp[pl.ds((g * 4 + 1) * 16, 16)][0]
        m2 = grp[pl.ds((g * 4 + 2) * 16, 16)][0]
        m3 = grp[pl.ds((g * 4 + 3) * 16, 16)][0]
        gm = jnp.maximum(jnp.maximum(m0, m1), jnp.maximum(m2, m3))
        win = jnp.where(m0 == gm, 0,
                        jnp.where(m1 == gm, 1,
                                  jnp.where(m2 == gm, 2, 3)))

        @pl.when(local == win)
        def _():
            bigv = jnp.full((16,), big, jnp.int32)

            def rchunk(k, acc):
                v = rmaxv[pl.ds(k * 16, 16)]
                return jnp.minimum(acc, jnp.where(v == gm, i16 + k * 16,
                                                  big))
            rloc = _vmin_s(lax.fori_loop(0, SROWS // 16, rchunk, bigv))

            def cchunk(k, acc):
                v = strip[pl.ds(rloc, 1), pl.ds(k * 16, 16)].reshape((16,))
                return jnp.minimum(acc, jnp.where(v == gm, i16 + k * 16,
                                                  big))
            col = _vmin_s(lax.fori_loop(0, W // 16, cchunk, bigv))
            yg = (r0 + rloc).astype(jnp.float32)
            xg = col.astype(jnp.float32)
            pkv[...] = jnp.where(i16 == 0, yg,
                                 jnp.where(i16 == 1, xg,
                                           jnp.full((16,), gm, jnp.float32)))
            pltpu.sync_copy(pkv, shared_pk.at[pl.ds(g * 16, 16)])
        plsc.subcore_barrier()

        pltpu.sync_copy(shared_pk, rdv)
        vpk = rdv[pl.ds(g * 16, 16)]
        ygf = vpk[0]
        xgf = vpk[1]
        gm2 = vpk[2]
        valid = gm2 > THRESH
        yi = ygf.astype(jnp.int32)
        xi = xgf.astype(jnp.int32)

        @pl.when(local == 0)
        def _():
            sm[2 * i] = jnp.where(valid, xgf, 0.0)
            sm[2 * i + 1] = jnp.where(valid, ygf, 0.0)

        @pl.when(valid)
        def _():
            ac = jnp.clip((xi - RADIUS) // 16, 0, W // 16 - 2)
            for k in range(2 * RADIUS + 1):
                ry = yi - RADIUS + k
                lr = ry - r0
                @pl.when((ry >= 0) & (ry < H) & (lr >= 0) & (lr < SROWS))
                def _():
                    for q in range(2):
                        off = pl.multiple_of((ac + q) * 16, 16)
                        vrow = strip[pl.ds(lr, 1),
                                     pl.ds(off, 16)].reshape((16,))
                        mask = jnp.abs(off + i16 - xi) <= RADIUS
                        strip[pl.ds(lr, 1), pl.ds(off, 16)] = jnp.where(
                            mask, 0.0, vrow).reshape((1, 16))
                    ms2 = _row_max(lr)
                    cb = pl.multiple_of((lr // 16) * 16, 16)
                    chv = rmaxv[pl.ds(cb, 16)]
                    rmaxv[pl.ds(cb, 16)] = jnp.where(
                        i16 == lr - cb, ms2, chv)
        return 0

    lax.fori_loop(0, NUM_PEAKS, iter_body, 0)

    @pl.when(local == 0)
    def _():
        def keyfill(j, _):
            px = sm[2 * j]
            py = sm[2 * j + 1]
            vj = (px + py) != 0.0
            sm[40 + j] = jnp.where(vj, py, NEG)
            return 0
        lax.fori_loop(0, NUM_PEAKS, keyfill, 0)

        ranks = []
        for ch in range(2):
            ioff = ch * 16
            kiv = jnp.full((16,), NEG, jnp.float32)
            for j2 in range(16):
                if ioff + j2 < NUM_PEAKS:
                    kiv = jnp.where(i16 == j2, sm[40 + ioff + j2], kiv)
            ig = i16 + ioff
            racc = jnp.zeros((16,), jnp.int32)
            for j in range(NUM_PEAKS):
                kj = sm[40 + j]
                a1 = jnp.where(kj > kiv, 1, 0)
                a2 = jnp.where(kj == kiv, 1, 0)
                a3 = jnp.where(j < ig, 1, 0)
                racc = racc + a1 + a2 * a3
            ranks.append(racc)

        ox0 = jnp.zeros((16,), jnp.float32)
        ox1 = jnp.zeros((16,), jnp.float32)
        oy0 = jnp.zeros((16,), jnp.float32)
        oy1 = jnp.zeros((16,), jnp.float32)
        for j in range(NUM_PEAKS):
            rj = ranks[j // 16][j % 16]
            xj = sm[2 * j]
            yj = sm[2 * j + 1]
            vj = (xj + yj) != 0.0
            xs = jnp.where(vj, xj, 0.0)
            ys = jnp.where(vj, yj, 0.0)
            ox0 = jnp.where(i16 == rj, xs, ox0)
            ox1 = jnp.where(i16 == rj - 16, xs, ox1)
            oy0 = jnp.where(i16 == rj, ys, oy0)
            oy1 = jnp.where(i16 == rj - 16, ys, oy1)
        ox[pl.ds(0, 16)] = ox0
        ox[pl.ds(16, 16)] = ox1
        oy[pl.ds(0, 16)] = oy0
        oy[pl.ds(16, 16)] = oy1
        pltpu.sync_copy(ox, out_hbm.at[m, 0])
        pltpu.sync_copy(oy, out_hbm.at[m, 1])


def _extract_all_peaks_sc(dual_hm):
    hm8 = dual_hm.reshape(2 * B, H, W)
    mesh = plsc.VectorSubcoreMesh(core_axis_name="c", subcore_axis_name="s")
    run = functools.partial(
        pl.kernel, mesh=mesh,
        out_type=jax.ShapeDtypeStruct((2 * B, 2, 32), jnp.float32),
        scratch_types=[
            pltpu.VMEM((SROWS, W), jnp.float32),
            pltpu.VMEM((SROWS,), jnp.float32),
            pltpu.VMEM((16,), jnp.float32),
            pltpu.VMEM((256,), jnp.float32),
            pltpu.VMEM((16,), jnp.float32),
            pltpu.VMEM((64,), jnp.float32),
            pltpu.VMEM((32,), jnp.float32),
            pltpu.VMEM((32,), jnp.float32),
            pltpu.VMEM_SHARED((256,), jnp.float32),
            pltpu.VMEM_SHARED((64,), jnp.float32),
            pltpu.SMEM((80,), jnp.float32),
        ],
    )(_sc_peaks_kernel)
    out8 = run(hm8)
    ou = jnp.transpose(out8[0::2, :, :NUM_PEAKS], (0, 2, 1))
    ol = jnp.transpose(out8[1::2, :, :NUM_PEAKS], (0, 2, 1))
    return ou, ol


def kernel(x, hm_w1, hm_b1, hm_g, hm_beta, hm_mean, hm_var, hm_w2, hm_b2,
           vec_w1, vec_b1, vec_g, vec_beta, vec_mean, vec_var, vec_w2,
           vec_b2):
    eps = 1e-5
    s_hm = hm_g / jnp.sqrt(hm_var + eps)
    s_vec = vec_g / jnp.sqrt(vec_var + eps)
    w1 = jnp.concatenate([hm_w1 * s_hm[:, None, None, None],
                          vec_w1 * s_vec[:, None, None, None]], axis=0)
    a = jnp.transpose(w1, (0, 2, 3, 1)).reshape(OC, 9 * C)
    b1 = jnp.concatenate([hm_b1 * s_hm + (hm_beta - hm_mean * s_hm),
                          vec_b1 * s_vec + (vec_beta - vec_mean * s_vec)])
    b1 = b1[:, None]
    w2 = jnp.zeros((8, OC), jnp.float32)
    w2 = w2.at[0:2, 0:HEAD].set(hm_w2.reshape(2, HEAD))
    w2 = w2.at[2:4, HEAD:OC].set(vec_w2.reshape(2, HEAD))
    b2 = jnp.zeros((8, 1), jnp.float32)
    b2 = b2.at[0:2, 0].set(hm_b2)
    b2 = b2.at[2:4, 0].set(vec_b2)

    dual_hm, vec_ind = _heads(x, a, b1, w2, b2)
    ordered_upper, ordered_lower = _extract_all_peaks_sc(dual_hm)
    mid = (ordered_upper + ordered_lower) / 2.0
    return (dual_hm, ordered_upper, ordered_lower, mid, vec_ind)


# speedup vs baseline: 4.0934x; 1.1802x over previous
import functools

import jax
import jax.numpy as jnp
from jax import lax
from jax.experimental import pallas as pl
from jax.experimental.pallas import tpu as pltpu
from jax.experimental.pallas import tpu_sc as plsc

B, C, H, W = 4, 32, 512, 512
HEAD = 64
RADIUS = 5
NUM_PEAKS = 18
THRESH = 0.1

R = 64
NR = H // R
OC = 2 * HEAD


def _conv_kernel(x_ref, halo_ref, a_ref, b1_ref, w2_ref, b2_ref,
                 hm_ref, vec_ref, scat):
    scat[C:2 * C, 1:R + 1, :] = x_ref[0]
    scat[C:2 * C, 0, :] = halo_ref[0, 0, 0]
    scat[C:2 * C, R + 1, :] = halo_ref[0, 0, 1]
    sv = scat[C:2 * C, :, :]
    scat[0:C, :, :] = jnp.pad(sv[:, :, :W - 1], ((0, 0), (0, 0), (1, 0)))
    scat[2 * C:3 * C, :, :] = jnp.pad(sv[:, :, 1:], ((0, 0), (0, 0), (0, 1)))

    for y in range(R):
        h = b1_ref[...]
        for dy in range(3):
            op = scat[:, y + dy, :]
            h = h + jnp.dot(a_ref[dy], op,
                            preferred_element_type=jnp.float32)
        h = jnp.maximum(h, 0.0)
        o = jnp.dot(w2_ref[...], h, preferred_element_type=jnp.float32)
        o = o + b2_ref[...]
        hm_ref[0, :, y, :] = jax.nn.sigmoid(o[0:2])
        vec_ref[0, :, y, :] = o[2:4]


def _heads(x, a, b1, w2, b2):
    z = jnp.zeros((B, C, 1, W), jnp.float32)
    tops = jnp.concatenate([z, x[:, :, R - 1:H - 1:R, :]], axis=2)
    bots = jnp.concatenate([x[:, :, R:H:R, :], z], axis=2)
    halo = jnp.stack([tops, bots], axis=2)
    halo = jnp.transpose(halo, (0, 3, 2, 1, 4))

    grid = (B, NR)
    out = pl.pallas_call(
        _conv_kernel,
        grid=grid,
        in_specs=[
            pl.BlockSpec((1, C, R, W), lambda b, j: (b, 0, j, 0)),
            pl.BlockSpec((1, 1, 2, C, W), lambda b, j: (b, j, 0, 0, 0)),
            pl.BlockSpec((3, OC, 3 * C), lambda b, j: (0, 0, 0)),
            pl.BlockSpec((OC, 1), lambda b, j: (0, 0)),
            pl.BlockSpec((8, OC), lambda b, j: (0, 0)),
            pl.BlockSpec((8, 1), lambda b, j: (0, 0)),
        ],
        out_specs=[
            pl.BlockSpec((1, 2, R, W), lambda b, j: (b, 0, j, 0)),
            pl.BlockSpec((1, 2, R, W), lambda b, j: (b, 0, j, 0)),
        ],
        out_shape=[
            jax.ShapeDtypeStruct((B, 2, H, W), jnp.float32),
            jax.ShapeDtypeStruct((B, 2, H, W), jnp.float32),
        ],
        scratch_shapes=[
            pltpu.VMEM((3 * C, R + 2, W), jnp.float32),
        ],
        compiler_params=pltpu.CompilerParams(
            dimension_semantics=("parallel", "parallel")),
    )(x, halo, a, b1, w2, b2)
    return out


WIN = 24


def _peaks_kernel(hm_ref, out_ref, hms, rmax):
    hm0 = hm_ref[0]
    hms[...] = hm0
    rmax[...] = jnp.max(hm0, axis=1, keepdims=True)

    iota_w = lax.broadcasted_iota(jnp.int32, (1, W), 1)
    iota_h = lax.broadcasted_iota(jnp.int32, (H, 1), 0)
    iota_8 = lax.broadcasted_iota(jnp.int32, (8, W), 0)
    big = jnp.int32(1 << 30)

    def body(i, peaks):
        rm = rmax[...]
        m = jnp.max(rm)
        y = jnp.min(jnp.where(rm == m, iota_h, big))
        ry = pl.multiple_of((y // 8) * 8, 8)
        rows8 = hms[pl.ds(ry, 8), :]
        rowv = jnp.max(jnp.where(iota_8 == (y - ry), rows8, -1.0),
                       axis=0, keepdims=True)
        x = jnp.min(jnp.where(rowv == m, iota_w, big))
        valid = m > THRESH

        gb = jnp.clip((y - RADIUS) // 8, 0, (H - WIN) // 8)
        yb = pl.multiple_of(gb * 8, 8)
        win = hms[pl.ds(yb, WIN), :]
        rr = yb + lax.broadcasted_iota(jnp.int32, (WIN, W), 0)
        cc = lax.broadcasted_iota(jnp.int32, (WIN, W), 1)
        sup = ((jnp.abs(rr - y) <= RADIUS) & (jnp.abs(cc - x) <= RADIUS)
               & valid)
        neww = jnp.where(sup, 0.0, win)
        hms[pl.ds(yb, WIN), :] = neww
        rmax[pl.ds(yb, WIN), :] = jnp.max(neww, axis=1, keepdims=True)

        rowm = lax.broadcasted_iota(jnp.int32, (NUM_PEAKS, 2), 0) == i
        colm = lax.broadcasted_iota(jnp.int32, (NUM_PEAKS, 2), 1)
        new = jnp.where(colm == 0, x.astype(jnp.float32),
                        y.astype(jnp.float32))
        return jnp.where(rowm & valid, new, peaks)

    peaks = lax.fori_loop(0, NUM_PEAKS, body,
                          jnp.zeros((NUM_PEAKS, 2), jnp.float32))

    px = peaks[:, 0]
    py = peaks[:, 1]
    validk = (px + py) != 0.0
    key = jnp.where(validk, py, -jnp.inf)
    jj = lax.broadcasted_iota(jnp.int32, (NUM_PEAKS, NUM_PEAKS), 0)
    ii = lax.broadcasted_iota(jnp.int32, (NUM_PEAKS, NUM_PEAKS), 1)
    kj = key[:, None]
    ki = key[None, :]
    rank = jnp.sum(((kj > ki) | ((kj == ki) & (jj < ii))).astype(jnp.int32),
                   axis=0)
    ptsm = jnp.where(validk[:, None], peaks, 0.0)
    onehot = (rank[:, None] ==
              lax.broadcasted_iota(jnp.int32, (NUM_PEAKS, NUM_PEAKS), 1))
    ordered = jnp.sum(onehot.astype(jnp.float32)[:, :, None]
                      * ptsm[:, None, :], axis=0)
    out_ref[0] = jnp.pad(ordered, ((0, 24 - NUM_PEAKS), (0, 126)))


def _extract_all_peaks(dual_hm):
    hm8 = dual_hm.reshape(2 * B, H, W)
    out = pl.pallas_call(
        _peaks_kernel,
        grid=(2 * B,),
        in_specs=[pl.BlockSpec((1, H, W), lambda i: (i, 0, 0))],
        out_specs=pl.BlockSpec((1, 24, 128), lambda i: (i, 0, 0)),
        out_shape=jax.ShapeDtypeStruct((2 * B, 24, 128), jnp.float32),
        scratch_shapes=[
            pltpu.VMEM((H, W), jnp.float32),
            pltpu.VMEM((H, 1), jnp.float32),
        ],
        compiler_params=pltpu.CompilerParams(
            dimension_semantics=("arbitrary",)),
    )(hm8)
    pk = out.reshape(B, 2, 24, 128)
    return pk[:, 0, :NUM_PEAKS, :2], pk[:, 1, :NUM_PEAKS, :2]


SROWS = H // 4
NEG = -3.0e38


def _vmax_s(v):
    m = v[0]
    for l in range(1, 16):
        m = jnp.maximum(m, v[l])
    return m


def _vmin_s(v):
    m = v[0]
    for l in range(1, 16):
        m = jnp.minimum(m, v[l])
    return m


def _sc_peaks_kernel(hm_hbm, out_hbm, strip, rmaxv, pub, grp, pkv, rdv,
                     ox, oy, shared_max, shared_pk, sm):
    c = lax.axis_index("c")
    s = lax.axis_index("s")
    g = s // 4
    local = s % 4
    r0 = local * SROWS
    m = c * 4 + g

    i16 = lax.broadcasted_iota(jnp.int32, (16,), 0)
    big = jnp.int32(1 << 30)
    negv = jnp.full((16,), NEG, jnp.float32)

    pltpu.sync_copy(hm_hbm.at[m, pl.ds(r0, SROWS), :], strip)

    def _row_max(r):
        def chunk(k, acc):
            v = strip[pl.ds(r, 1), pl.ds(k * 16, 16)].reshape((16,))
            return jnp.maximum(acc, v)
        return _vmax_s(lax.fori_loop(0, W // 16, chunk, negv))

    def init_group(rg, _):
        def row_in(j, accv):
            ms = _row_max(rg * 16 + j)
            return jnp.where(i16 == j, ms, accv)
        rmaxv[pl.ds(rg * 16, 16)] = lax.fori_loop(
            0, 16, row_in, jnp.zeros((16,), jnp.float32))
        return 0

    lax.fori_loop(0, SROWS // 16, init_group, 0)

    def iter_body(i, _):
        def chunk8(k, acc):
            return jnp.maximum(acc, rmaxv[pl.ds(k * 16, 16)])
        sm_t = _vmax_s(lax.fori_loop(0, SROWS // 16, chunk8, negv))
        pub[...] = jnp.full((16,), sm_t, jnp.float32)
        pltpu.sync_copy(pub, shared_max.at[pl.ds(s * 16, 16)])
        plsc.subcore_barrier()

        pltpu.sync_copy(shared_max, grp)
        m0 = grp[pl.ds((g * 4 + 0) * 16, 16)][0]
        m1 = grp[pl.ds((g * 4 + 1) * 16, 16)][0]
        m2 = grp[pl.ds((g * 4 + 2) * 16, 16)][0]
        m3 = grp[pl.ds((g * 4 + 3) * 16, 16)][0]
        gm = jnp.maximum(jnp.maximum(m0, m1), jnp.maximum(m2, m3))
        win = jnp.where(m0 == gm, 0,
                        jnp.where(m1 == gm, 1,
                                  jnp.where(m2 == gm, 2, 3)))

        @pl.when(local == win)
        def _():
            bigv = jnp.full((16,), big, jnp.int32)

            def rchunk(k, acc):
                v = rmaxv[pl.ds(k * 16, 16)]
                return jnp.minimum(acc, jnp.where(v == gm, i16 + k * 16,
                                                  big))
            rloc = _vmin_s(lax.fori_loop(0, SROWS // 16, rchunk, bigv))

            def cchunk(k, acc):
                v = strip[pl.ds(rloc, 1), pl.ds(k * 16, 16)].reshape((16,))
                return jnp.minimum(acc, jnp.where(v == gm, i16 + k * 16,
                                                  big))
            col = _vmin_s(lax.fori_loop(0, W // 16, cchunk, bigv))
            yg = (r0 + rloc).astype(jnp.float32)
            xg = col.astype(jnp.float32)
            pkv[...] = jnp.where(i16 == 0, yg,
                                 jnp.where(i16 == 1, xg,
                                           jnp.full((16,), gm, jnp.float32)))
            pltpu.sync_copy(pkv, shared_pk.at[pl.ds(g * 16, 16)])
        plsc.subcore_barrier()

        pltpu.sync_copy(shared_pk, rdv)
        vpk = rdv[pl.ds(g * 16, 16)]
        ygf = vpk[0]
        xgf = vpk[1]
        gm2 = vpk[2]
        valid = gm2 > THRESH
        yi = ygf.astype(jnp.int32)
        xi = xgf.astype(jnp.int32)

        @pl.when(local == 0)
        def _():
            sm[2 * i] = jnp.where(valid, xgf, 0.0)
            sm[2 * i + 1] = jnp.where(valid, ygf, 0.0)

        @pl.when(valid)
        def _():
            ac = jnp.clip((xi - RADIUS) // 16, 0, W // 16 - 2)
            for k in range(2 * RADIUS + 1):
                ry = yi - RADIUS + k
                lr = ry - r0
                @pl.when((ry >= 0) & (ry < H) & (lr >= 0) & (lr < SROWS))
                def _():
                    for q in range(2):
                        off = pl.multiple_of((ac + q) * 16, 16)
                        vrow = strip[pl.ds(lr, 1),
                                     pl.ds(off, 16)].reshape((16,))
                        mask = jnp.abs(off + i16 - xi) <= RADIUS
                        strip[pl.ds(lr, 1), pl.ds(off, 16)] = jnp.where(
                            mask, 0.0, vrow).reshape((1, 16))
                    ms2 = _row_max(lr)
                    cb = pl.multiple_of((lr // 16) * 16, 16)
                    chv = rmaxv[pl.ds(cb, 16)]
                    rmaxv[pl.ds(cb, 16)] = jnp.where(
                        i16 == lr - cb, ms2, chv)
        return 0

    lax.fori_loop(0, NUM_PEAKS, iter_body, 0)

    @pl.when(local == 0)
    def _():
        def keyfill(j, _):
            px = sm[2 * j]
            py = sm[2 * j + 1]
            vj = (px + py) != 0.0
            sm[40 + j] = jnp.where(vj, py, NEG)
            return 0
        lax.fori_loop(0, NUM_PEAKS, keyfill, 0)

        ranks = []
        for ch in range(2):
            ioff = ch * 16
            kiv = jnp.full((16,), NEG, jnp.float32)
            for j2 in range(16):
                if ioff + j2 < NUM_PEAKS:
                    kiv = jnp.where(i16 == j2, sm[40 + ioff + j2], kiv)
            ig = i16 + ioff
            racc = jnp.zeros((16,), jnp.int32)
            for j in range(NUM_PEAKS):
                kj = sm[40 + j]
                a1 = jnp.where(kj > kiv, 1, 0)
                a2 = jnp.where(kj == kiv, 1, 0)
                a3 = jnp.where(j < ig, 1, 0)
                racc = racc + a1 + a2 * a3
            ranks.append(racc)

        ox0 = jnp.zeros((16,), jnp.float32)
        ox1 = jnp.zeros((16,), jnp.float32)
        oy0 = jnp.zeros((16,), jnp.float32)
        oy1 = jnp.zeros((16,), jnp.float32)
        for j in range(NUM_PEAKS):
            rj = ranks[j // 16][j % 16]
            xj = sm[2 * j]
            yj = sm[2 * j + 1]
            vj = (xj + yj) != 0.0
            xs = jnp.where(vj, xj, 0.0)
            ys = jnp.where(vj, yj, 0.0)
            ox0 = jnp.where(i16 == rj, xs, ox0)
            ox1 = jnp.where(i16 == rj - 16, xs, ox1)
            oy0 = jnp.where(i16 == rj, ys, oy0)
            oy1 = jnp.where(i16 == rj - 16, ys, oy1)
        ox[pl.ds(0, 16)] = ox0
        ox[pl.ds(16, 16)] = ox1
        oy[pl.ds(0, 16)] = oy0
        oy[pl.ds(16, 16)] = oy1
        pltpu.sync_copy(ox, out_hbm.at[m, 0])
        pltpu.sync_copy(oy, out_hbm.at[m, 1])


def _extract_all_peaks_sc(dual_hm):
    hm8 = dual_hm.reshape(2 * B, H, W)
    mesh = plsc.VectorSubcoreMesh(core_axis_name="c", subcore_axis_name="s")
    run = functools.partial(
        pl.kernel, mesh=mesh,
        out_type=jax.ShapeDtypeStruct((2 * B, 2, 32), jnp.float32),
        scratch_types=[
            pltpu.VMEM((SROWS, W), jnp.float32),
            pltpu.VMEM((SROWS,), jnp.float32),
            pltpu.VMEM((16,), jnp.float32),
            pltpu.VMEM((256,), jnp.float32),
            pltpu.VMEM((16,), jnp.float32),
            pltpu.VMEM((64,), jnp.float32),
            pltpu.VMEM((32,), jnp.float32),
            pltpu.VMEM((32,), jnp.float32),
            pltpu.VMEM_SHARED((256,), jnp.float32),
            pltpu.VMEM_SHARED((64,), jnp.float32),
            pltpu.SMEM((80,), jnp.float32),
        ],
    )(_sc_peaks_kernel)
    out8 = run(hm8)
    ou = jnp.transpose(out8[0::2, :, :NUM_PEAKS], (0, 2, 1))
    ol = jnp.transpose(out8[1::2, :, :NUM_PEAKS], (0, 2, 1))
    return ou, ol


def kernel(x, hm_w1, hm_b1, hm_g, hm_beta, hm_mean, hm_var, hm_w2, hm_b2,
           vec_w1, vec_b1, vec_g, vec_beta, vec_mean, vec_var, vec_w2,
           vec_b2):
    eps = 1e-5
    s_hm = hm_g / jnp.sqrt(hm_var + eps)
    s_vec = vec_g / jnp.sqrt(vec_var + eps)
    w1 = jnp.concatenate([hm_w1 * s_hm[:, None, None, None],
                          vec_w1 * s_vec[:, None, None, None]], axis=0)
    a = jnp.transpose(w1, (2, 0, 3, 1)).reshape(3, OC, 3 * C)
    b1 = jnp.concatenate([hm_b1 * s_hm + (hm_beta - hm_mean * s_hm),
                          vec_b1 * s_vec + (vec_beta - vec_mean * s_vec)])
    b1 = b1[:, None]
    w2 = jnp.zeros((8, OC), jnp.float32)
    w2 = w2.at[0:2, 0:HEAD].set(hm_w2.reshape(2, HEAD))
    w2 = w2.at[2:4, HEAD:OC].set(vec_w2.reshape(2, HEAD))
    b2 = jnp.zeros((8, 1), jnp.float32)
    b2 = b2.at[0:2, 0].set(hm_b2)
    b2 = b2.at[2:4, 0].set(vec_b2)

    dual_hm, vec_ind = _heads(x, a, b1, w2, b2)
    ordered_upper, ordered_lower = _extract_all_peaks_sc(dual_hm)
    mid = (ordered_upper + ordered_lower) / 2.0
    return (dual_hm, ordered_upper, ordered_lower, mid, vec_ind)


# speedup vs baseline: 6.0416x; 1.4760x over previous
import functools

import jax
import jax.numpy as jnp
from jax import lax
from jax.experimental import pallas as pl
from jax.experimental.pallas import tpu as pltpu
from jax.experimental.pallas import tpu_sc as plsc

B, C, H, W = 4, 32, 512, 512
HEAD = 64
RADIUS = 5
NUM_PEAKS = 18
THRESH = 0.1

R = 64
NR = H // R
OC = 2 * HEAD


def _conv_kernel(x_ref, halo_ref, a_ref, b1_ref, w2_ref, b2_ref,
                 hm_ref, vec_ref, scat):
    scat[C:2 * C, 1:R + 1, :] = x_ref[0]
    scat[C:2 * C, 0, :] = halo_ref[0, 0, 0]
    scat[C:2 * C, R + 1, :] = halo_ref[0, 0, 1]
    sv = scat[C:2 * C, :, :]
    scat[0:C, :, :] = jnp.pad(sv[:, :, :W - 1], ((0, 0), (0, 0), (1, 0)))
    scat[2 * C:3 * C, :, :] = jnp.pad(sv[:, :, 1:], ((0, 0), (0, 0), (0, 1)))

    G = 4
    for y in range(0, R, G):
        h = b1_ref[...]
        for dy in range(3):
            op = jnp.concatenate(
                [scat[:, y + i + dy, :] for i in range(G)], axis=1)
            h = h + jnp.dot(a_ref[dy], op,
                            preferred_element_type=jnp.float32)
        h = jnp.maximum(h, 0.0)
        o = jnp.dot(w2_ref[...], h, preferred_element_type=jnp.float32)
        o = o + b2_ref[...]
        sig = jax.nn.sigmoid(o[0:2])
        for i in range(G):
            hm_ref[0, :, y + i, :] = sig[:, i * W:(i + 1) * W]
            vec_ref[0, :, y + i, :] = o[2:4, i * W:(i + 1) * W]


def _heads(x, a, b1, w2, b2):
    z = jnp.zeros((B, C, 1, W), jnp.float32)
    tops = jnp.concatenate([z, x[:, :, R - 1:H - 1:R, :]], axis=2)
    bots = jnp.concatenate([x[:, :, R:H:R, :], z], axis=2)
    halo = jnp.stack([tops, bots], axis=2)
    halo = jnp.transpose(halo, (0, 3, 2, 1, 4))

    grid = (B, NR)
    out = pl.pallas_call(
        _conv_kernel,
        grid=grid,
        in_specs=[
            pl.BlockSpec((1, C, R, W), lambda b, j: (b, 0, j, 0)),
            pl.BlockSpec((1, 1, 2, C, W), lambda b, j: (b, j, 0, 0, 0)),
            pl.BlockSpec((3, OC, 3 * C), lambda b, j: (0, 0, 0)),
            pl.BlockSpec((OC, 1), lambda b, j: (0, 0)),
            pl.BlockSpec((8, OC), lambda b, j: (0, 0)),
            pl.BlockSpec((8, 1), lambda b, j: (0, 0)),
        ],
        out_specs=[
            pl.BlockSpec((1, 2, R, W), lambda b, j: (b, 0, j, 0)),
            pl.BlockSpec((1, 2, R, W), lambda b, j: (b, 0, j, 0)),
        ],
        out_shape=[
            jax.ShapeDtypeStruct((B, 2, H, W), jnp.float32),
            jax.ShapeDtypeStruct((B, 2, H, W), jnp.float32),
        ],
        scratch_shapes=[
            pltpu.VMEM((3 * C, R + 2, W), jnp.float32),
        ],
        compiler_params=pltpu.CompilerParams(
            dimension_semantics=("parallel", "parallel")),
    )(x, halo, a, b1, w2, b2)
    return out


WIN = 24


def _peaks_kernel(hm_ref, out_ref, hms, rmax):
    hm0 = hm_ref[0]
    hms[...] = hm0
    rmax[...] = jnp.max(hm0, axis=1, keepdims=True)

    iota_w = lax.broadcasted_iota(jnp.int32, (1, W), 1)
    iota_h = lax.broadcasted_iota(jnp.int32, (H, 1), 0)
    iota_8 = lax.broadcasted_iota(jnp.int32, (8, W), 0)
    big = jnp.int32(1 << 30)

    def body(i, peaks):
        rm = rmax[...]
        m = jnp.max(rm)
        y = jnp.min(jnp.where(rm == m, iota_h, big))
        ry = pl.multiple_of((y // 8) * 8, 8)
        rows8 = hms[pl.ds(ry, 8), :]
        rowv = jnp.max(jnp.where(iota_8 == (y - ry), rows8, -1.0),
                       axis=0, keepdims=True)
        x = jnp.min(jnp.where(rowv == m, iota_w, big))
        valid = m > THRESH

        gb = jnp.clip((y - RADIUS) // 8, 0, (H - WIN) // 8)
        yb = pl.multiple_of(gb * 8, 8)
        win = hms[pl.ds(yb, WIN), :]
        rr = yb + lax.broadcasted_iota(jnp.int32, (WIN, W), 0)
        cc = lax.broadcasted_iota(jnp.int32, (WIN, W), 1)
        sup = ((jnp.abs(rr - y) <= RADIUS) & (jnp.abs(cc - x) <= RADIUS)
               & valid)
        neww = jnp.where(sup, 0.0, win)
        hms[pl.ds(yb, WIN), :] = neww
        rmax[pl.ds(yb, WIN), :] = jnp.max(neww, axis=1, keepdims=True)

        rowm = lax.broadcasted_iota(jnp.int32, (NUM_PEAKS, 2), 0) == i
        colm = lax.broadcasted_iota(jnp.int32, (NUM_PEAKS, 2), 1)
        new = jnp.where(colm == 0, x.astype(jnp.float32),
                        y.astype(jnp.float32))
        return jnp.where(rowm & valid, new, peaks)

    peaks = lax.fori_loop(0, NUM_PEAKS, body,
                          jnp.zeros((NUM_PEAKS, 2), jnp.float32))

    px = peaks[:, 0]
    py = peaks[:, 1]
    validk = (px + py) != 0.0
    key = jnp.where(validk, py, -jnp.inf)
    jj = lax.broadcasted_iota(jnp.int32, (NUM_PEAKS, NUM_PEAKS), 0)
    ii = lax.broadcasted_iota(jnp.int32, (NUM_PEAKS, NUM_PEAKS), 1)
    kj = key[:, None]
    ki = key[None, :]
    rank = jnp.sum(((kj > ki) | ((kj == ki) & (jj < ii))).astype(jnp.int32),
                   axis=0)
    ptsm = jnp.where(validk[:, None], peaks, 0.0)
    onehot = (rank[:, None] ==
              lax.broadcasted_iota(jnp.int32, (NUM_PEAKS, NUM_PEAKS), 1))
    ordered = jnp.sum(onehot.astype(jnp.float32)[:, :, None]
                      * ptsm[:, None, :], axis=0)
    out_ref[0] = jnp.pad(ordered, ((0, 24 - NUM_PEAKS), (0, 126)))


def _extract_all_peaks(dual_hm):
    hm8 = dual_hm.reshape(2 * B, H, W)
    out = pl.pallas_call(
        _peaks_kernel,
        grid=(2 * B,),
        in_specs=[pl.BlockSpec((1, H, W), lambda i: (i, 0, 0))],
        out_specs=pl.BlockSpec((1, 24, 128), lambda i: (i, 0, 0)),
        out_shape=jax.ShapeDtypeStruct((2 * B, 24, 128), jnp.float32),
        scratch_shapes=[
            pltpu.VMEM((H, W), jnp.float32),
            pltpu.VMEM((H, 1), jnp.float32),
        ],
        compiler_params=pltpu.CompilerParams(
            dimension_semantics=("arbitrary",)),
    )(hm8)
    pk = out.reshape(B, 2, 24, 128)
    return pk[:, 0, :NUM_PEAKS, :2], pk[:, 1, :NUM_PEAKS, :2]


SROWS = H // 4
NEG = -3.0e38


def _vmax_s(v):
    m = v[0]
    for l in range(1, 16):
        m = jnp.maximum(m, v[l])
    return m


def _vmin_s(v):
    m = v[0]
    for l in range(1, 16):
        m = jnp.minimum(m, v[l])
    return m


def _sc_peaks_kernel(hm_hbm, out_hbm, strip, rmaxv, pub, grp, pkv, rdv,
                     ox, oy, shared_max, shared_pk, sm):
    c = lax.axis_index("c")
    s = lax.axis_index("s")
    g = s // 4
    local = s % 4
    r0 = local * SROWS
    m = c * 4 + g

    i16 = lax.broadcasted_iota(jnp.int32, (16,), 0)
    big = jnp.int32(1 << 30)
    negv = jnp.full((16,), NEG, jnp.float32)

    pltpu.sync_copy(hm_hbm.at[m, pl.ds(r0, SROWS), :], strip)

    def _row_max(r):
        def chunk(k, acc):
            v = strip[pl.ds(r, 1), pl.ds(k * 16, 16)].reshape((16,))
            return jnp.maximum(acc, v)
        return _vmax_s(lax.fori_loop(0, W // 16, chunk, negv))

    def init_group(rg, _):
        def row_in(j, accv):
            ms = _row_max(rg * 16 + j)
            return jnp.where(i16 == j, ms, accv)
        rmaxv[pl.ds(rg * 16, 16)] = lax.fori_loop(
            0, 16, row_in, jnp.zeros((16,), jnp.float32))
        return 0

    lax.fori_loop(0, SROWS // 16, init_group, 0)

    def iter_body(i, _):
        def chunk8(k, acc):
            return jnp.maximum(acc, rmaxv[pl.ds(k * 16, 16)])
        sm_t = _vmax_s(lax.fori_loop(0, SROWS // 16, chunk8, negv))
        pub[...] = jnp.full((16,), sm_t, jnp.float32)
        pltpu.sync_copy(pub, shared_max.at[pl.ds(s * 16, 16)])
        plsc.subcore_barrier()

        pltpu.sync_copy(shared_max, grp)
        m0 = grp[pl.ds((g * 4 + 0) * 16, 16)][0]
        m1 = grp[pl.ds((g * 4 + 1) * 16, 16)][0]
        m2 = grp[pl.ds((g * 4 + 2) * 16, 16)][0]
        m3 = grp[pl.ds((g * 4 + 3) * 16, 16)][0]
        gm = jnp.maximum(jnp.maximum(m0, m1), jnp.maximum(m2, m3))
        win = jnp.where(m0 == gm, 0,
                        jnp.where(m1 == gm, 1,
                                  jnp.where(m2 == gm, 2, 3)))

        @pl.when(local == win)
        def _():
            bigv = jnp.full((16,), big, jnp.int32)

            def rchunk(k, acc):
                v = rmaxv[pl.ds(k * 16, 16)]
                return jnp.minimum(acc, jnp.where(v == gm, i16 + k * 16,
                                                  big))
            rloc = _vmin_s(lax.fori_loop(0, SROWS // 16, rchunk, bigv))

            def cchunk(k, acc):
                v = strip[pl.ds(rloc, 1), pl.ds(k * 16, 16)].reshape((16,))
                return jnp.minimum(acc, jnp.where(v == gm, i16 + k * 16,
                                                  big))
            col = _vmin_s(lax.fori_loop(0, W // 16, cchunk, bigv))
            yg = (r0 + rloc).astype(jnp.float32)
            xg = col.astype(jnp.float32)
            pkv[...] = jnp.where(i16 == 0, yg,
                                 jnp.where(i16 == 1, xg,
                                           jnp.full((16,), gm, jnp.float32)))
            pltpu.sync_copy(pkv, shared_pk.at[pl.ds(g * 16, 16)])
        plsc.subcore_barrier()

        pltpu.sync_copy(shared_pk, rdv)
        vpk = rdv[pl.ds(g * 16, 16)]
        ygf = vpk[0]
        xgf = vpk[1]
        gm2 = vpk[2]
        valid = gm2 > THRESH
        yi = ygf.astype(jnp.int32)
        xi = xgf.astype(jnp.int32)

        @pl.when(local == 0)
        def _():
            sm[2 * i] = jnp.where(valid, xgf, 0.0)
            sm[2 * i + 1] = jnp.where(valid, ygf, 0.0)

        @pl.when(valid)
        def _():
            ac = jnp.clip((xi - RADIUS) // 16, 0, W // 16 - 2)
            for k in range(2 * RADIUS + 1):
                ry = yi - RADIUS + k
                lr = ry - r0
                @pl.when((ry >= 0) & (ry < H) & (lr >= 0) & (lr < SROWS))
                def _():
                    for q in range(2):
                        off = pl.multiple_of((ac + q) * 16, 16)
                        vrow = strip[pl.ds(lr, 1),
                                     pl.ds(off, 16)].reshape((16,))
                        mask = jnp.abs(off + i16 - xi) <= RADIUS
                        strip[pl.ds(lr, 1), pl.ds(off, 16)] = jnp.where(
                            mask, 0.0, vrow).reshape((1, 16))
                    ms2 = _row_max(lr)
                    cb = pl.multiple_of((lr // 16) * 16, 16)
                    chv = rmaxv[pl.ds(cb, 16)]
                    rmaxv[pl.ds(cb, 16)] = jnp.where(
                        i16 == lr - cb, ms2, chv)
        return 0

    lax.fori_loop(0, NUM_PEAKS, iter_body, 0)

    @pl.when(local == 0)
    def _():
        def keyfill(j, _):
            px = sm[2 * j]
            py = sm[2 * j + 1]
            vj = (px + py) != 0.0
            sm[40 + j] = jnp.where(vj, py, NEG)
            return 0
        lax.fori_loop(0, NUM_PEAKS, keyfill, 0)

        ranks = []
        for ch in range(2):
            ioff = ch * 16
            kiv = jnp.full((16,), NEG, jnp.float32)
            for j2 in range(16):
                if ioff + j2 < NUM_PEAKS:
                    kiv = jnp.where(i16 == j2, sm[40 + ioff + j2], kiv)
            ig = i16 + ioff
            racc = jnp.zeros((16,), jnp.int32)
            for j in range(NUM_PEAKS):
                kj = sm[40 + j]
                a1 = jnp.where(kj > kiv, 1, 0)
                a2 = jnp.where(kj == kiv, 1, 0)
                a3 = jnp.where(j < ig, 1, 0)
                racc = racc + a1 + a2 * a3
            ranks.append(racc)

        ox0 = jnp.zeros((16,), jnp.float32)
        ox1 = jnp.zeros((16,), jnp.float32)
        oy0 = jnp.zeros((16,), jnp.float32)
        oy1 = jnp.zeros((16,), jnp.float32)
        for j in range(NUM_PEAKS):
            rj = ranks[j // 16][j % 16]
            xj = sm[2 * j]
            yj = sm[2 * j + 1]
            vj = (xj + yj) != 0.0
            xs = jnp.where(vj, xj, 0.0)
            ys = jnp.where(vj, yj, 0.0)
            ox0 = jnp.where(i16 == rj, xs, ox0)
            ox1 = jnp.where(i16 == rj - 16, xs, ox1)
            oy0 = jnp.where(i16 == rj, ys, oy0)
            oy1 = jnp.where(i16 == rj - 16, ys, oy1)
        ox[pl.ds(0, 16)] = ox0
        ox[pl.ds(16, 16)] = ox1
        oy[pl.ds(0, 16)] = oy0
        oy[pl.ds(16, 16)] = oy1
        pltpu.sync_copy(ox, out_hbm.at[m, 0])
        pltpu.sync_copy(oy, out_hbm.at[m, 1])


def _extract_all_peaks_sc(dual_hm):
    hm8 = dual_hm.reshape(2 * B, H, W)
    mesh = plsc.VectorSubcoreMesh(core_axis_name="c", subcore_axis_name="s")
    run = functools.partial(
        pl.kernel, mesh=mesh,
        out_type=jax.ShapeDtypeStruct((2 * B, 2, 32), jnp.float32),
        scratch_types=[
            pltpu.VMEM((SROWS, W), jnp.float32),
            pltpu.VMEM((SROWS,), jnp.float32),
            pltpu.VMEM((16,), jnp.float32),
            pltpu.VMEM((256,), jnp.float32),
            pltpu.VMEM((16,), jnp.float32),
            pltpu.VMEM((64,), jnp.float32),
            pltpu.VMEM((32,), jnp.float32),
            pltpu.VMEM((32,), jnp.float32),
            pltpu.VMEM_SHARED((256,), jnp.float32),
            pltpu.VMEM_SHARED((64,), jnp.float32),
            pltpu.SMEM((80,), jnp.float32),
        ],
    )(_sc_peaks_kernel)
    out8 = run(hm8)
    ou = jnp.transpose(out8[0::2, :, :NUM_PEAKS], (0, 2, 1))
    ol = jnp.transpose(out8[1::2, :, :NUM_PEAKS], (0, 2, 1))
    return ou, ol


def kernel(x, hm_w1, hm_b1, hm_g, hm_beta, hm_mean, hm_var, hm_w2, hm_b2,
           vec_w1, vec_b1, vec_g, vec_beta, vec_mean, vec_var, vec_w2,
           vec_b2):
    eps = 1e-5
    s_hm = hm_g / jnp.sqrt(hm_var + eps)
    s_vec = vec_g / jnp.sqrt(vec_var + eps)
    w1 = jnp.concatenate([hm_w1 * s_hm[:, None, None, None],
                          vec_w1 * s_vec[:, None, None, None]], axis=0)
    a = jnp.transpose(w1, (2, 0, 3, 1)).reshape(3, OC, 3 * C)
    b1 = jnp.concatenate([hm_b1 * s_hm + (hm_beta - hm_mean * s_hm),
                          vec_b1 * s_vec + (vec_beta - vec_mean * s_vec)])
    b1 = b1[:, None]
    w2 = jnp.zeros((8, OC), jnp.float32)
    w2 = w2.at[0:2, 0:HEAD].set(hm_w2.reshape(2, HEAD))
    w2 = w2.at[2:4, HEAD:OC].set(vec_w2.reshape(2, HEAD))
    b2 = jnp.zeros((8, 1), jnp.float32)
    b2 = b2.at[0:2, 0].set(hm_b2)
    b2 = b2.at[2:4, 0].set(vec_b2)

    dual_hm, vec_ind = _heads(x, a, b1, w2, b2)
    ordered_upper, ordered_lower = _extract_all_peaks_sc(dual_hm)
    mid = (ordered_upper + ordered_lower) / 2.0
    return (dual_hm, ordered_upper, ordered_lower, mid, vec_ind)
